# Initial kernel scaffold; baseline (speedup 1.0000x reference)
import jax
import jax.numpy as jnp
from jax.experimental import pallas as pl


def kernel(traj_input, mat_input, traj_length, W_t, W_l, W_u, W_su, W_sl, W_tu, W_tl):
    raise NotImplementedError("write your pallas kernel here")


# speedup vs baseline: 14.1469x; 14.1469x over previous
import functools

import jax
import jax.numpy as jnp
from jax import lax
from jax.experimental import pallas as pl
from jax.experimental.pallas import tpu as pltpu
from jax.experimental.pallas import tpu_sc as plsc

B, L, D = 1024, 20, 64
HOURS = 24 * 7
NC, NS = 2, 16
NW = NC * NS
ROWS = B * L
RPW = ROWS // NW
GCHUNK = 128
NCHUNK = RPW // GCHUNK


def _sc_joint_body(wt_hbm, wl_hbm, wu_hbm, uidx_hbm, lidx_hbm, traw_hbm,
                   out_hbm, uidx_v, lidx_v, tidx_v, acc_v, tmp_v, sem):
    wid = lax.axis_index("s") * NC + lax.axis_index("c")
    base = wid * RPW
    pltpu.sync_copy(uidx_hbm.at[pl.ds(base, RPW)], uidx_v)
    pltpu.sync_copy(lidx_hbm.at[pl.ds(base, RPW)], lidx_v)
    pltpu.sync_copy(traw_hbm.at[pl.ds(base, RPW)], tidx_v)

    def fix_t(i, _):
        x = tidx_v[pl.ds(i * 16, 16)]
        tidx_v[pl.ds(i * 16, 16)] = (x + (HOURS - 1)) % HOURS + 1
        return 0

    lax.fori_loop(0, RPW // 16, fix_t, 0, unroll=4)

    def gather(table, idx_v, dst_v):
        cps = []
        for k in range(NCHUNK):
            cps.append(pltpu.async_copy(
                table.at[idx_v.at[pl.ds(k * GCHUNK, GCHUNK)]],
                dst_v.at[pl.ds(k * GCHUNK, GCHUNK), :], sem))
        for cp in cps:
            cp.wait()

    def accumulate():
        def addrow(r, _):
            for c in range(D // 16):
                acc_v[r, pl.ds(c * 16, 16)] = (
                    acc_v[r, pl.ds(c * 16, 16)] + tmp_v[r, pl.ds(c * 16, 16)])
            return 0
        lax.fori_loop(0, RPW, addrow, 0, unroll=4)

    gather(wu_hbm, uidx_v, acc_v)
    gather(wl_hbm, lidx_v, tmp_v)
    accumulate()
    gather(wt_hbm, tidx_v, tmp_v)
    accumulate()
    pltpu.sync_copy(acc_v, out_hbm.at[pl.ds(base, RPW)])


@functools.partial(jax.jit, static_argnames=())
def _sc_joint(W_t, W_l, W_u, u_idx, l_idx, t_raw):
    mesh = plsc.VectorSubcoreMesh(core_axis_name="c", subcore_axis_name="s")
    return pl.kernel(
        _sc_joint_body,
        out_type=jax.ShapeDtypeStruct((ROWS, D), jnp.float32),
        mesh=mesh,
        scratch_types=[
            pltpu.VMEM((RPW,), jnp.int32),
            pltpu.VMEM((RPW,), jnp.int32),
            pltpu.VMEM((RPW,), jnp.int32),
            pltpu.VMEM((RPW, D), jnp.float32),
            pltpu.VMEM((RPW, D), jnp.float32),
            pltpu.SemaphoreType.DMA,
        ],
        compiler_params=pltpu.CompilerParams(use_tc_tiling_on_sc=False),
    )(W_t, W_l, W_u, u_idx, l_idx, t_raw)


BB = 32


def _tc_delta_body(len_ref, ds_ref, dt_ref, wsu_ref, wsl_ref, wtu_ref,
                   wtl_ref, out_ref):
    wsl = wsl_ref[:, :]
    wsu = wsu_ref[:, :]
    wtl = wtl_ref[:, :]
    wtu = wtu_ref[:, :]
    basev = wsl + wtl
    svec = wsu - wsl
    tvec = wtu - wtl

    p = lax.broadcasted_iota(jnp.int32, (BB, L * L), 1)
    i = p // L
    j = p - i * L
    ln = len_ref[:, :]
    m = ((i < ln) & (j < ln)).astype(jnp.float32)[:, :, None]

    ds = ds_ref[:, :][:, :, None]
    dt = dt_ref[:, :][:, :, None]
    out_ref[:, :, :] = (
        basev[0] + m * (basev[1] - basev[0])
        + ds * (svec[0] + m * (svec[1] - svec[0]))
        + dt * (tvec[0] + m * (tvec[1] - tvec[0])))


def _tc_delta(traj_length2d, ds, dt, W_su, W_sl, W_tu, W_tl):
    grid = (B // BB,)
    return pl.pallas_call(
        _tc_delta_body,
        grid=grid,
        in_specs=[
            pl.BlockSpec((BB, 1), lambda b: (b, 0)),
            pl.BlockSpec((BB, L * L), lambda b: (b, 0)),
            pl.BlockSpec((BB, L * L), lambda b: (b, 0)),
            pl.BlockSpec((2, D), lambda b: (0, 0)),
            pl.BlockSpec((2, D), lambda b: (0, 0)),
            pl.BlockSpec((2, D), lambda b: (0, 0)),
            pl.BlockSpec((2, D), lambda b: (0, 0)),
        ],
        out_specs=pl.BlockSpec((BB, L * L, D), lambda b: (b, 0, 0)),
        out_shape=jax.ShapeDtypeStruct((B, L * L, D), jnp.float32),
        compiler_params=pltpu.CompilerParams(
            dimension_semantics=("arbitrary",)),
    )(traj_length2d, ds, dt, W_su, W_sl, W_tu, W_tl)


def kernel(traj_input, mat_input, traj_length, W_t, W_l, W_u, W_su, W_sl,
           W_tu, W_tl):
    u_idx = traj_input[:, :, 0].reshape(ROWS)
    l_idx = traj_input[:, :, 1].reshape(ROWS)
    t_raw = traj_input[:, :, 2].reshape(ROWS)

    joint = _sc_joint(W_t, W_l, W_u, u_idx, l_idx, t_raw).reshape(B, L, D)

    ds = mat_input[:, :, :, 0].reshape(B, L * L)
    dt = mat_input[:, :, :, 1].reshape(B, L * L)
    delta = _tc_delta(traj_length.reshape(B, 1), ds, dt, W_su, W_sl, W_tu,
                      W_tl).reshape(B, L, L, D)
    return (joint, delta)
